# Initial kernel scaffold; baseline (speedup 1.0000x reference)
import jax
import jax.numpy as jnp
from jax.experimental import pallas as pl


def kernel(x, batch, W1, W2):
    raise NotImplementedError("write your pallas kernel here")


# speedup vs baseline: 6.1923x; 6.1923x over previous
import functools

import jax
import jax.numpy as jnp
from jax import lax
from jax.experimental import pallas as pl
from jax.experimental.pallas import tpu as pltpu

N = 100000
C = 256
G = 64
H = 16

BLK = 2000
NBLK = N // BLK


def _pool_mlp_kernel(x_ref, b_ref, w1_ref, w2_ref, scale_ref, acc_ref, cnt_ref):
    i = pl.program_id(0)

    @pl.when(i == 0)
    def _init():
        acc_ref[...] = jnp.zeros_like(acc_ref)
        cnt_ref[...] = jnp.zeros_like(cnt_ref)

    seg = b_ref[0, 0, :]
    gids = lax.broadcasted_iota(jnp.int32, (G, BLK), 0)
    onehot = (gids == seg[None, :]).astype(jnp.float32)
    acc_ref[...] += jax.lax.dot_general(
        onehot, x_ref[...], (((1,), (0,)), ((), ())),
        preferred_element_type=jnp.float32)
    cnt_ref[...] += jnp.sum(onehot, axis=1, keepdims=True)

    @pl.when(i == NBLK - 1)
    def _finish():
        counts = jnp.maximum(cnt_ref[...], 1.0)
        mean = acc_ref[...] / counts
        h = jax.lax.dot_general(mean, w1_ref[...], (((1,), (1,)), ((), ())),
                                preferred_element_type=jnp.float32)
        h = jnp.maximum(h, 0.0)
        logits = jax.lax.dot_general(h, w2_ref[...], (((1,), (1,)), ((), ())),
                                     preferred_element_type=jnp.float32)
        scale_ref[...] = jax.nn.sigmoid(logits)


def _scale_kernel(x_ref, b_ref, scale_ref, out_ref):
    seg = b_ref[0, 0, :]
    gids = lax.broadcasted_iota(jnp.int32, (BLK, G), 1)
    onehot = (gids == seg[:, None]).astype(jnp.float32)
    rows = jax.lax.dot_general(onehot, scale_ref[...], (((1,), (0,)), ((), ())),
                               preferred_element_type=jnp.float32)
    out_ref[...] = x_ref[...] * rows


def kernel(x, batch, W1, W2):
    b32 = batch.astype(jnp.int32).reshape(NBLK, 1, BLK)

    scale = pl.pallas_call(
        _pool_mlp_kernel,
        grid=(NBLK,),
        in_specs=[
            pl.BlockSpec((BLK, C), lambda i: (i, 0)),
            pl.BlockSpec((1, 1, BLK), lambda i: (i, 0, 0)),
            pl.BlockSpec((H, C), lambda i: (0, 0)),
            pl.BlockSpec((C, H), lambda i: (0, 0)),
        ],
        out_specs=pl.BlockSpec((G, C), lambda i: (0, 0)),
        out_shape=jax.ShapeDtypeStruct((G, C), jnp.float32),
        scratch_shapes=[
            pltpu.VMEM((G, C), jnp.float32),
            pltpu.VMEM((G, 1), jnp.float32),
        ],
    )(x, b32, W1, W2)

    out = pl.pallas_call(
        _scale_kernel,
        grid=(NBLK,),
        in_specs=[
            pl.BlockSpec((BLK, C), lambda i: (i, 0)),
            pl.BlockSpec((1, 1, BLK), lambda i: (i, 0, 0)),
            pl.BlockSpec((G, C), lambda i: (0, 0)),
        ],
        out_specs=pl.BlockSpec((BLK, C), lambda i: (i, 0)),
        out_shape=jax.ShapeDtypeStruct((N, C), jnp.float32),
    )(x, b32, scale)
    return out


# speedup vs baseline: 7.5949x; 1.2265x over previous
import functools

import jax
import jax.numpy as jnp
from jax import lax
from jax.experimental import pallas as pl
from jax.experimental.pallas import tpu as pltpu

N = 100000
C = 256
G = 64
H = 16

BLK = 4000
NBLK = N // BLK


def _pool_mlp_kernel(x_ref, b_ref, w1_ref, w2_ref, scale_ref, acc_ref, cnt_ref):
    i = pl.program_id(0)

    @pl.when(i == 0)
    def _init():
        acc_ref[...] = jnp.zeros_like(acc_ref)
        cnt_ref[...] = jnp.zeros_like(cnt_ref)

    seg = b_ref[0, 0, :]
    gids = lax.broadcasted_iota(jnp.int32, (G, BLK), 0)
    onehot = (gids == seg[None, :]).astype(jnp.float32)
    acc_ref[...] += jax.lax.dot_general(
        onehot, x_ref[...], (((1,), (0,)), ((), ())),
        preferred_element_type=jnp.float32)
    cnt_ref[...] += jnp.sum(onehot, axis=1, keepdims=True)

    @pl.when(i == NBLK - 1)
    def _finish():
        counts = jnp.maximum(cnt_ref[...], 1.0)
        mean = acc_ref[...] / counts
        h = jax.lax.dot_general(mean, w1_ref[...], (((1,), (1,)), ((), ())),
                                preferred_element_type=jnp.float32)
        h = jnp.maximum(h, 0.0)
        logits = jax.lax.dot_general(h, w2_ref[...], (((1,), (1,)), ((), ())),
                                     preferred_element_type=jnp.float32)
        scale_ref[...] = jax.nn.sigmoid(logits)


def _scale_kernel(x_ref, b_ref, scale_ref, out_ref):
    seg = b_ref[0, 0, :]
    gids = lax.broadcasted_iota(jnp.int32, (BLK, G), 1)
    onehot = (gids == seg[:, None]).astype(jnp.float32)
    rows = jax.lax.dot_general(onehot, scale_ref[...], (((1,), (0,)), ((), ())),
                               preferred_element_type=jnp.float32)
    out_ref[...] = x_ref[...] * rows


def kernel(x, batch, W1, W2):
    b32 = batch.astype(jnp.int32).reshape(NBLK, 1, BLK)

    scale = pl.pallas_call(
        _pool_mlp_kernel,
        grid=(NBLK,),
        in_specs=[
            pl.BlockSpec((BLK, C), lambda i: (i, 0)),
            pl.BlockSpec((1, 1, BLK), lambda i: (i, 0, 0)),
            pl.BlockSpec((H, C), lambda i: (0, 0)),
            pl.BlockSpec((C, H), lambda i: (0, 0)),
        ],
        out_specs=pl.BlockSpec((G, C), lambda i: (0, 0)),
        out_shape=jax.ShapeDtypeStruct((G, C), jnp.float32),
        scratch_shapes=[
            pltpu.VMEM((G, C), jnp.float32),
            pltpu.VMEM((G, 1), jnp.float32),
        ],
    )(x, b32, W1, W2)

    out = pl.pallas_call(
        _scale_kernel,
        grid=(NBLK,),
        in_specs=[
            pl.BlockSpec((BLK, C), lambda i: (i, 0)),
            pl.BlockSpec((1, 1, BLK), lambda i: (i, 0, 0)),
            pl.BlockSpec((G, C), lambda i: (0, 0)),
        ],
        out_specs=pl.BlockSpec((BLK, C), lambda i: (i, 0)),
        out_shape=jax.ShapeDtypeStruct((N, C), jnp.float32),
    )(x, b32, scale)
    return out


# speedup vs baseline: 7.9067x; 1.0410x over previous
import functools

import jax
import jax.numpy as jnp
from jax import lax
from jax.experimental import pallas as pl
from jax.experimental.pallas import tpu as pltpu

N = 100000
C = 256
G = 64
H = 16

BLK = 5000
NBLK = N // BLK


def _pool_mlp_kernel(x_ref, b_ref, w1_ref, w2_ref, scale_ref, acc_ref, cnt_ref):
    i = pl.program_id(0)

    @pl.when(i == 0)
    def _init():
        acc_ref[...] = jnp.zeros_like(acc_ref)
        cnt_ref[...] = jnp.zeros_like(cnt_ref)

    seg = b_ref[0, 0, :]
    gids = lax.broadcasted_iota(jnp.int32, (G, BLK), 0)
    onehot = (gids == seg[None, :]).astype(jnp.float32)
    acc_ref[...] += jax.lax.dot_general(
        onehot, x_ref[...], (((1,), (0,)), ((), ())),
        preferred_element_type=jnp.float32)
    cnt_ref[...] += jnp.sum(onehot, axis=1, keepdims=True)

    @pl.when(i == NBLK - 1)
    def _finish():
        counts = jnp.maximum(cnt_ref[...], 1.0)
        mean = acc_ref[...] / counts
        h = jax.lax.dot_general(mean, w1_ref[...], (((1,), (1,)), ((), ())),
                                preferred_element_type=jnp.float32)
        h = jnp.maximum(h, 0.0)
        logits = jax.lax.dot_general(h, w2_ref[...], (((1,), (1,)), ((), ())),
                                     preferred_element_type=jnp.float32)
        scale_ref[...] = jax.nn.sigmoid(logits)


def _scale_kernel(x_ref, b_ref, scale_ref, out_ref):
    seg = b_ref[0, 0, :]
    gids = lax.broadcasted_iota(jnp.int32, (BLK, G), 1)
    onehot = (gids == seg[:, None]).astype(jnp.float32)
    rows = jax.lax.dot_general(onehot, scale_ref[...], (((1,), (0,)), ((), ())),
                               preferred_element_type=jnp.float32)
    out_ref[...] = x_ref[...] * rows


def kernel(x, batch, W1, W2):
    b32 = batch.astype(jnp.int32).reshape(NBLK, 1, BLK)

    scale = pl.pallas_call(
        _pool_mlp_kernel,
        grid=(NBLK,),
        in_specs=[
            pl.BlockSpec((BLK, C), lambda i: (i, 0)),
            pl.BlockSpec((1, 1, BLK), lambda i: (i, 0, 0)),
            pl.BlockSpec((H, C), lambda i: (0, 0)),
            pl.BlockSpec((C, H), lambda i: (0, 0)),
        ],
        out_specs=pl.BlockSpec((G, C), lambda i: (0, 0)),
        out_shape=jax.ShapeDtypeStruct((G, C), jnp.float32),
        scratch_shapes=[
            pltpu.VMEM((G, C), jnp.float32),
            pltpu.VMEM((G, 1), jnp.float32),
        ],
    )(x, b32, W1, W2)

    out = pl.pallas_call(
        _scale_kernel,
        grid=(NBLK,),
        in_specs=[
            pl.BlockSpec((BLK, C), lambda i: (i, 0)),
            pl.BlockSpec((1, 1, BLK), lambda i: (i, 0, 0)),
            pl.BlockSpec((G, C), lambda i: (0, 0)),
        ],
        out_specs=pl.BlockSpec((BLK, C), lambda i: (i, 0)),
        out_shape=jax.ShapeDtypeStruct((N, C), jnp.float32),
    )(x, b32, scale)
    return out


# speedup vs baseline: 8.1730x; 1.0337x over previous
import functools

import jax
import jax.numpy as jnp
from jax import lax
from jax.experimental import pallas as pl
from jax.experimental.pallas import tpu as pltpu

N = 100000
C = 256
G = 64
H = 16

BLK = 10000
NBLK = N // BLK


def _pool_mlp_kernel(x_ref, b_ref, w1_ref, w2_ref, scale_ref, acc_ref, cnt_ref):
    i = pl.program_id(0)

    @pl.when(i == 0)
    def _init():
        acc_ref[...] = jnp.zeros_like(acc_ref)
        cnt_ref[...] = jnp.zeros_like(cnt_ref)

    seg = b_ref[0, 0, :]
    gids = lax.broadcasted_iota(jnp.int32, (G, BLK), 0)
    onehot = (gids == seg[None, :]).astype(jnp.float32)
    acc_ref[...] += jax.lax.dot_general(
        onehot, x_ref[...], (((1,), (0,)), ((), ())),
        preferred_element_type=jnp.float32)
    cnt_ref[...] += jnp.sum(onehot, axis=1, keepdims=True)

    @pl.when(i == NBLK - 1)
    def _finish():
        counts = jnp.maximum(cnt_ref[...], 1.0)
        mean = acc_ref[...] / counts
        h = jax.lax.dot_general(mean, w1_ref[...], (((1,), (1,)), ((), ())),
                                preferred_element_type=jnp.float32)
        h = jnp.maximum(h, 0.0)
        logits = jax.lax.dot_general(h, w2_ref[...], (((1,), (1,)), ((), ())),
                                     preferred_element_type=jnp.float32)
        scale_ref[...] = jax.nn.sigmoid(logits)


def _scale_kernel(x_ref, b_ref, scale_ref, out_ref):
    seg = b_ref[0, 0, :]
    gids = lax.broadcasted_iota(jnp.int32, (BLK, G), 1)
    onehot = (gids == seg[:, None]).astype(jnp.float32)
    rows = jax.lax.dot_general(onehot, scale_ref[...], (((1,), (0,)), ((), ())),
                               preferred_element_type=jnp.float32)
    out_ref[...] = x_ref[...] * rows


def kernel(x, batch, W1, W2):
    b32 = batch.astype(jnp.int32).reshape(NBLK, 1, BLK)

    scale = pl.pallas_call(
        _pool_mlp_kernel,
        grid=(NBLK,),
        in_specs=[
            pl.BlockSpec((BLK, C), lambda i: (i, 0)),
            pl.BlockSpec((1, 1, BLK), lambda i: (i, 0, 0)),
            pl.BlockSpec((H, C), lambda i: (0, 0)),
            pl.BlockSpec((C, H), lambda i: (0, 0)),
        ],
        out_specs=pl.BlockSpec((G, C), lambda i: (0, 0)),
        out_shape=jax.ShapeDtypeStruct((G, C), jnp.float32),
        scratch_shapes=[
            pltpu.VMEM((G, C), jnp.float32),
            pltpu.VMEM((G, 1), jnp.float32),
        ],
    )(x, b32, W1, W2)

    out = pl.pallas_call(
        _scale_kernel,
        grid=(NBLK,),
        in_specs=[
            pl.BlockSpec((BLK, C), lambda i: (i, 0)),
            pl.BlockSpec((1, 1, BLK), lambda i: (i, 0, 0)),
            pl.BlockSpec((G, C), lambda i: (0, 0)),
        ],
        out_specs=pl.BlockSpec((BLK, C), lambda i: (i, 0)),
        out_shape=jax.ShapeDtypeStruct((N, C), jnp.float32),
    )(x, b32, scale)
    return out
